# baseline (speedup 1.0000x reference)
import functools

import jax
import jax.numpy as jnp
from jax import lax
from jax.experimental import pallas as pl
from jax.experimental.pallas import tpu as pltpu
from jax.experimental.pallas import tpu_sc as plsc

VOCAB = 1000000
EMBD = 32
B = 4096
L = 200

NC = 2
NS = 16
NW = NC * NS

BT = B // 128
LB = L // 8
KU = 4
NGRP = L // KU


def _make_kernel():
    mesh = plsc.VectorSubcoreMesh(core_axis_name="c", subcore_axis_name="s")

    @functools.partial(
        pl.kernel,
        out_type=jax.ShapeDtypeStruct((L, EMBD // 8, BT, 8, 128), jnp.float32),
        mesh=mesh,
        compiler_params=pltpu.CompilerParams(
            use_tc_tiling_on_sc=False, needs_layout_passes=False
        ),
        scratch_types=[
            pltpu.VMEM((LB, 8, 128), jnp.int32),
            pltpu.VMEM((KU * 128, EMBD), jnp.float32),
            pltpu.VMEM((KU * 128, EMBD), jnp.float32),
            pltpu.VMEM((KU, EMBD // 8, 8, 128), jnp.float32),
            pltpu.VMEM((KU, EMBD // 8, 8, 128), jnp.float32),
            pltpu.SemaphoreType.DMA,
            pltpu.SemaphoreType.DMA,
            pltpu.SemaphoreType.DMA,
            pltpu.SemaphoreType.DMA,
        ],
    )
    def emb_kernel(table_hbm, x4_hbm, out_hbm,
                   idx_v, gb0, gb1, tb0, tb1, g0, g1, s0, s1):
        gbuf = (gb0, gb1)
        tbuf = (tb0, tb1)
        gsem = (g0, g1)
        ssem = (s0, s1)
        wid = lax.axis_index("s") * NC + lax.axis_index("c")
        pltpu.sync_copy(x4_hbm.at[:, wid], idx_v)

        def gather_start(g, s):
            for u in range(KU):
                l = g * KU + u
                pltpu.async_copy(
                    table_hbm.at[idx_v.at[l // 8, l % 8]],
                    gbuf[s].at[pl.ds(u * 128, 128)],
                    gsem[s],
                )

        def gather_wait(g, s):
            for u in range(KU):
                l = g * KU + u
                pltpu.make_async_copy(
                    table_hbm.at[idx_v.at[l // 8, l % 8]],
                    gbuf[s].at[pl.ds(u * 128, 128)],
                    gsem[s],
                ).wait()

        def store_start(g, s):
            pltpu.async_copy(
                tbuf[s], out_hbm.at[pl.ds(g * KU, KU), :, wid], ssem[s]
            )

        def store_wait(g, s):
            pltpu.make_async_copy(
                tbuf[s], out_hbm.at[pl.ds(g * KU, KU), :, wid], ssem[s]
            ).wait()

        def transpose_relu(s):
            src = gbuf[s]
            dst = tbuf[s]
            lanes = lax.iota(jnp.int32, 16)
            for u in range(KU):

                @pl.loop(0, 8)
                def _b16(b16):
                    rows = lanes + (u * 128) + b16 * 16
                    for e in range(EMBD):
                        cols = jnp.full((16,), e, jnp.int32)
                        vals = plsc.load_gather(src, [rows, cols])
                        dst[u, e // 8, e % 8, pl.ds(b16 * 16, 16)] = (
                            jnp.maximum(vals, 0.0)
                        )

        gather_start(0, 0)

        @pl.loop(0, NGRP, step=2)
        def _pair(G):
            for s in range(2):
                g = G + s
                o = 1 - s

                @pl.when(g >= 1)
                def _drain_prev_store():
                    store_wait(g - 1, o)

                @pl.when(g + 1 < NGRP)
                def _fire_next_gather():
                    gather_start(g + 1, o)

                gather_wait(g, s)
                transpose_relu(s)
                store_start(g, s)

        store_wait(NGRP - 1, 1)

    return emb_kernel


_EMB_KERNEL = _make_kernel()


@jax.jit
def kernel(x, table):
    x4 = (
        x.astype(jnp.int32)
        .T.reshape(LB, 8, BT, 128)
        .transpose(0, 2, 1, 3)
    )
    out5 = _EMB_KERNEL(table, x4)
    return out5.transpose(2, 4, 0, 1, 3).reshape(B, L, EMBD)


# speedup vs baseline: 1.1864x; 1.1864x over previous
import functools

import jax
import jax.numpy as jnp
from jax import lax
from jax.experimental import pallas as pl
from jax.experimental.pallas import tpu as pltpu
from jax.experimental.pallas import tpu_sc as plsc

VOCAB = 1000000
EMBD = 32
B = 4096
L = 200

NC = 2
NS = 16
NW = NC * NS

XPAD = 203
KU = 5
NGRP = L // KU


def _make_kernel():
    mesh = plsc.VectorSubcoreMesh(core_axis_name="c", subcore_axis_name="s")

    @functools.partial(
        pl.kernel,
        out_type=jax.ShapeDtypeStruct((B, L, EMBD), jnp.float32),
        mesh=mesh,
        compiler_params=pltpu.CompilerParams(
            use_tc_tiling_on_sc=False, needs_layout_passes=False
        ),
        scratch_types=[
            pltpu.VMEM((128, XPAD), jnp.int32),
            pltpu.VMEM((L, 128), jnp.int32),
            pltpu.VMEM((KU * 128, EMBD), jnp.float32),
            pltpu.VMEM((KU * 128, EMBD), jnp.float32),
            pltpu.SemaphoreType.DMA,
            pltpu.SemaphoreType.DMA,
            pltpu.SemaphoreType.DMA,
            pltpu.SemaphoreType.DMA,
        ],
    )
    def emb_kernel(table_hbm, x_hbm, out_hbm,
                   xbuf, idx_v, gb0, gb1, g0, g1, s0, s1):
        gbuf = (gb0, gb1)
        gsem = (g0, g1)
        ssem = (s0, s1)
        wid = lax.axis_index("s") * NC + lax.axis_index("c")
        b0 = wid * 128
        lanes = lax.iota(jnp.int32, 16)

        pltpu.sync_copy(x_hbm.at[pl.ds(b0, 128), :], xbuf.at[:, pl.ds(0, L)])

        @pl.loop(0, L)
        def _tr(l):
            cols = jnp.full((16,), l, jnp.int32)
            for k in range(8):
                vals = plsc.load_gather(xbuf, [lanes + k * 16, cols])
                idx_v[l, pl.ds(k * 16, 16)] = vals

        def gather_start(g, s):
            for u in range(KU):
                pltpu.async_copy(
                    table_hbm.at[idx_v.at[g * KU + u]],
                    gbuf[s].at[pl.ds(u * 128, 128)],
                    gsem[s],
                )

        def gather_wait(g, s):
            for u in range(KU):
                pltpu.make_async_copy(
                    table_hbm.at[idx_v.at[g * KU + u]],
                    gbuf[s].at[pl.ds(u * 128, 128)],
                    gsem[s],
                ).wait()

        def store_start(g, s):
            for u in range(KU):
                pltpu.async_copy(
                    gbuf[s].at[pl.ds(u * 128, 128)],
                    out_hbm.at[pl.ds(b0, 128), g * KU + u],
                    ssem[s],
                )

        def store_wait(g, s):
            for u in range(KU):
                pltpu.make_async_copy(
                    gbuf[s].at[pl.ds(u * 128, 128)],
                    out_hbm.at[pl.ds(b0, 128), g * KU + u],
                    ssem[s],
                ).wait()

        def relu(s):
            buf = gbuf[s]

            @pl.loop(0, KU * 128, unroll=4)
            def _rows(i):
                buf[i, 0:16] = jnp.maximum(buf[i, 0:16], 0.0)
                buf[i, 16:32] = jnp.maximum(buf[i, 16:32], 0.0)

        gather_start(0, 0)

        @pl.loop(0, NGRP, step=2)
        def _pair(G):
            for s in range(2):
                g = G + s
                o = 1 - s

                @pl.when(g >= 1)
                def _drain_prev_store():
                    store_wait(g - 1, o)

                @pl.when(g + 1 < NGRP)
                def _fire_next_gather():
                    gather_start(g + 1, o)

                gather_wait(g, s)
                relu(s)
                store_start(g, s)

        store_wait(NGRP - 1, 1)

    return emb_kernel


_EMB_KERNEL = _make_kernel()


@jax.jit
def kernel(x, table):
    return _EMB_KERNEL(table, x.astype(jnp.int32))


# speedup vs baseline: 1.5222x; 1.2830x over previous
import functools

import jax
import jax.numpy as jnp
from jax import lax
from jax.experimental import pallas as pl
from jax.experimental.pallas import tpu as pltpu
from jax.experimental.pallas import tpu_sc as plsc

VOCAB = 1000000
EMBD = 32
B = 4096
L = 200

NC = 2
NS = 16
NW = NC * NS

BT = B // 128
XPAD = 203
TPAD = 131
KU = 4
NGRP = L // KU


def _make_kernel():
    mesh = plsc.VectorSubcoreMesh(core_axis_name="c", subcore_axis_name="s")

    @functools.partial(
        pl.kernel,
        out_type=jax.ShapeDtypeStruct((L, EMBD // 8, BT, 8, 128), jnp.float32),
        mesh=mesh,
        compiler_params=pltpu.CompilerParams(
            use_tc_tiling_on_sc=False, needs_layout_passes=False
        ),
        scratch_types=[
            pltpu.VMEM((128, XPAD), jnp.float32),
            pltpu.VMEM((L, 128), jnp.int32),
            pltpu.VMEM((KU * 128, EMBD), jnp.float32),
            pltpu.VMEM((KU * 128, EMBD), jnp.float32),
            pltpu.VMEM((KU, EMBD // 8, 8, TPAD), jnp.float32),
            pltpu.VMEM((KU, EMBD // 8, 8, TPAD), jnp.float32),
            pltpu.SemaphoreType.DMA,
            pltpu.SemaphoreType.DMA,
            pltpu.SemaphoreType.DMA,
            pltpu.SemaphoreType.DMA,
        ],
    )
    def emb_kernel(table_hbm, xf_hbm, out_hbm,
                   xbuf, idx_v, gb0, gb1, tb0, tb1, g0, g1, s0, s1):
        gbuf = (gb0, gb1)
        tbuf = (tb0, tb1)
        gsem = (g0, g1)
        ssem = (s0, s1)
        wid = lax.axis_index("s") * NC + lax.axis_index("c")
        b0 = wid * 128
        lanes = lax.iota(jnp.int32, 16)

        pltpu.sync_copy(xf_hbm.at[pl.ds(b0, 128), :], xbuf.at[:, pl.ds(0, L)])

        @pl.loop(0, L)
        def _tr(l):
            cols = jnp.full((16,), l, jnp.int32)
            for k in range(8):
                vals = plsc.load_gather(xbuf, [lanes + k * 16, cols])
                idx_v[l, pl.ds(k * 16, 16)] = plsc.bitcast(vals, jnp.int32)

        def gather_start(g, s):
            for u in range(KU):
                pltpu.async_copy(
                    table_hbm.at[idx_v.at[g * KU + u]],
                    gbuf[s].at[pl.ds(u * 128, 128)],
                    gsem[s],
                )

        def gather_wait(g, s):
            for u in range(KU):
                pltpu.make_async_copy(
                    table_hbm.at[idx_v.at[g * KU + u]],
                    gbuf[s].at[pl.ds(u * 128, 128)],
                    gsem[s],
                ).wait()

        def store_start(g, s):
            pltpu.async_copy(
                tbuf[s].at[:, :, :, pl.ds(0, 128)],
                out_hbm.at[pl.ds(g * KU, KU), :, wid],
                ssem[s],
            )

        def store_wait(g, s):
            pltpu.make_async_copy(
                tbuf[s].at[:, :, :, pl.ds(0, 128)],
                out_hbm.at[pl.ds(g * KU, KU), :, wid],
                ssem[s],
            ).wait()

        esub = lanes & 7
        band0 = lanes >> 3
        band1 = band0 + 2

        def transpose_relu(s):
            src = gbuf[s]
            dst = tbuf[s]
            for u in range(KU):
                ub = jnp.full((16,), u, jnp.int32)

                @pl.loop(0, 128, unroll=2)
                def _row(r):
                    rb = jnp.full((16,), r, jnp.int32)
                    row = u * 128 + r
                    v0 = jnp.maximum(src[row, 0:16], 0.0)
                    plsc.store_scatter(dst, [ub, band0, esub, rb], v0)
                    v1 = jnp.maximum(src[row, 16:32], 0.0)
                    plsc.store_scatter(dst, [ub, band1, esub, rb], v1)

        gather_start(0, 0)

        @pl.loop(0, NGRP, step=2)
        def _pair(G):
            for s in range(2):
                g = G + s
                o = 1 - s

                @pl.when(g >= 1)
                def _drain_prev_store():
                    store_wait(g - 1, o)

                @pl.when(g + 1 < NGRP)
                def _fire_next_gather():
                    gather_start(g + 1, o)

                gather_wait(g, s)
                transpose_relu(s)
                store_start(g, s)

        store_wait(NGRP - 1, 1)

    return emb_kernel


_EMB_KERNEL = _make_kernel()


@jax.jit
def kernel(x, table):
    xf = jax.lax.bitcast_convert_type(x.astype(jnp.int32), jnp.float32)
    out5 = _EMB_KERNEL(table, xf)
    return out5.transpose(2, 4, 0, 1, 3).reshape(B, L, EMBD)


# speedup vs baseline: 1.5499x; 1.0182x over previous
import functools

import jax
import jax.numpy as jnp
from jax import lax
from jax.experimental import pallas as pl
from jax.experimental.pallas import tpu as pltpu
from jax.experimental.pallas import tpu_sc as plsc

VOCAB = 1000000
EMBD = 32
B = 4096
L = 200

NC = 2
NS = 16
NW = NC * NS

BT = B // 128
LB = L // 8
TPAD = 131
KU = 5
NGRP = L // KU


def _make_formatter():
    mesh = plsc.VectorSubcoreMesh(core_axis_name="c", subcore_axis_name="s")

    @functools.partial(
        pl.kernel,
        out_type=jax.ShapeDtypeStruct((LB, BT, 8, 128), jnp.int32),
        mesh=mesh,
        compiler_params=pltpu.CompilerParams(use_tc_tiling_on_sc=True),
        scratch_types=[
            pltpu.VMEM((LB, 8, 128), jnp.int32),
        ],
    )
    def fmt_kernel(xt_hbm, out_hbm, buf):
        wid = lax.axis_index("s") * NC + lax.axis_index("c")
        for lb in range(LB):
            pltpu.sync_copy(
                xt_hbm.at[pl.ds(lb * 8, 8), pl.ds(wid * 128, 128)],
                buf.at[lb],
            )
        pltpu.sync_copy(buf, out_hbm.at[:, wid])

    return fmt_kernel


def _make_kernel():
    mesh = plsc.VectorSubcoreMesh(core_axis_name="c", subcore_axis_name="s")

    @functools.partial(
        pl.kernel,
        out_type=jax.ShapeDtypeStruct((L, EMBD // 8, BT, 8, 128), jnp.float32),
        mesh=mesh,
        compiler_params=pltpu.CompilerParams(
            use_tc_tiling_on_sc=False, needs_layout_passes=False
        ),
        scratch_types=[
            pltpu.VMEM((LB, 8, 128), jnp.int32),
            pltpu.VMEM((KU * 128, EMBD), jnp.float32),
            pltpu.VMEM((KU * 128, EMBD), jnp.float32),
            pltpu.VMEM((KU, EMBD // 8, 8, TPAD), jnp.float32),
            pltpu.VMEM((KU, EMBD // 8, 8, TPAD), jnp.float32),
            pltpu.SemaphoreType.DMA,
            pltpu.SemaphoreType.DMA,
            pltpu.SemaphoreType.DMA,
            pltpu.SemaphoreType.DMA,
        ],
    )
    def emb_kernel(table_hbm, x4_hbm, out_hbm,
                   idx_v, gb0, gb1, tb0, tb1, g0, g1, s0, s1):
        gbuf = (gb0, gb1)
        tbuf = (tb0, tb1)
        gsem = (g0, g1)
        ssem = (s0, s1)
        wid = lax.axis_index("s") * NC + lax.axis_index("c")
        lanes = lax.iota(jnp.int32, 16)

        pltpu.sync_copy(x4_hbm.at[:, wid], idx_v)

        def gather_start(g, s):
            for u in range(KU):
                l = g * KU + u
                pltpu.async_copy(
                    table_hbm.at[idx_v.at[l // 8, l % 8]],
                    gbuf[s].at[pl.ds(u * 128, 128)],
                    gsem[s],
                )

        def gather_wait(g, s):
            for u in range(KU):
                l = g * KU + u
                pltpu.make_async_copy(
                    table_hbm.at[idx_v.at[l // 8, l % 8]],
                    gbuf[s].at[pl.ds(u * 128, 128)],
                    gsem[s],
                ).wait()

        def store_start(g, s):
            pltpu.async_copy(
                tbuf[s].at[:, :, :, pl.ds(0, 128)],
                out_hbm.at[pl.ds(g * KU, KU), :, wid],
                ssem[s],
            )

        def store_wait(g, s):
            pltpu.make_async_copy(
                tbuf[s].at[:, :, :, pl.ds(0, 128)],
                out_hbm.at[pl.ds(g * KU, KU), :, wid],
                ssem[s],
            ).wait()

        esub = lanes & 7
        band0 = lanes >> 3
        band1 = band0 + 2

        def transpose_relu(s):
            src = gbuf[s]
            dst = tbuf[s]
            for u in range(KU):
                ub = jnp.full((16,), u, jnp.int32)

                @pl.loop(0, 128, unroll=2)
                def _row(r):
                    rb = jnp.full((16,), r, jnp.int32)
                    row = u * 128 + r
                    v0 = jnp.maximum(src[row, 0:16], 0.0)
                    plsc.store_scatter(dst, [ub, band0, esub, rb], v0)
                    v1 = jnp.maximum(src[row, 16:32], 0.0)
                    plsc.store_scatter(dst, [ub, band1, esub, rb], v1)

        gather_start(0, 0)

        @pl.loop(0, NGRP, step=2)
        def _pair(G):
            for s in range(2):
                g = G + s
                o = 1 - s

                @pl.when(g >= 1)
                def _drain_prev_store():
                    store_wait(g - 1, o)

                @pl.when(g + 1 < NGRP)
                def _fire_next_gather():
                    gather_start(g + 1, o)

                gather_wait(g, s)
                transpose_relu(s)
                store_start(g, s)

        store_wait(NGRP - 1, 1)

    return emb_kernel


_FMT_KERNEL = _make_formatter()
_EMB_KERNEL = _make_kernel()


@jax.jit
def kernel(x, table):
    x4 = _FMT_KERNEL(x.astype(jnp.int32).T)
    out5 = _EMB_KERNEL(table, x4)
    return out5.transpose(2, 4, 0, 1, 3).reshape(B, L, EMBD)
